# baseline (speedup 1.0000x reference)
import functools

import jax
import jax.numpy as jnp
from jax import lax
from jax.experimental import pallas as pl
from jax.experimental.pallas import tpu as pltpu
from jax.experimental.pallas import tpu_sc as plsc

_B = 1024
_M = 8192
_R = 4
_NC = 2
_NS = 16
_L = 16
_NW = _NC * _NS
_KSC = 256
_ROWS = _KSC // _NW
_NBUF = 2
_BBLK = 64


def _freeness_sc(ww, rw, u, fg):
    mesh = plsc.VectorSubcoreMesh(
        core_axis_name="c", subcore_axis_name="s",
        num_cores=_NC, num_subcores=_NS,
    )

    @functools.partial(
        pl.kernel,
        out_type=jax.ShapeDtypeStruct((_KSC, _M), jnp.float32),
        mesh=mesh,
        scratch_types=[
            pltpu.VMEM((_ROWS * _R + _L,), jnp.float32),
            pltpu.VMEM((_NBUF, _M), jnp.float32),
            pltpu.VMEM((_NBUF, _M), jnp.float32),
            pltpu.VMEM((_NBUF, _R, _M), jnp.float32),
            pltpu.VMEM((_NBUF, _M), jnp.float32),
            pltpu.SemaphoreType.DMA,
            pltpu.SemaphoreType.DMA,
            pltpu.SemaphoreType.DMA,
            pltpu.SemaphoreType.DMA,
        ],
    )
    def body(ww_hbm, rw_hbm, u_hbm, fg_hbm, out_hbm,
             fg_v, u_v, w_v, rw_v, o_v, si0, si1, so0, so1):
        wid = lax.axis_index("s") * _NC + lax.axis_index("c")
        base = wid * _ROWS
        pltpu.sync_copy(fg_hbm.at[pl.ds(base * _R, _ROWS * _R)],
                        fg_v.at[pl.ds(0, _ROWS * _R)])
        sin = (si0, si1)
        sout = (so0, so1)

        def issue_in(r, s):
            b = base + r
            cps = (
                pltpu.make_async_copy(u_hbm.at[b], u_v.at[s], sin[s]),
                pltpu.make_async_copy(ww_hbm.at[b, 0], w_v.at[s], sin[s]),
                pltpu.make_async_copy(rw_hbm.at[b], rw_v.at[s], sin[s]),
            )
            for cp in cps:
                cp.start()
            return cps

        def compute(r, s):
            vv = fg_v[pl.ds(r * _R, _L)]
            f0 = jnp.full((_L,), vv[0], jnp.float32)
            f1 = jnp.full((_L,), vv[1], jnp.float32)
            f2 = jnp.full((_L,), vv[2], jnp.float32)
            f3 = jnp.full((_L,), vv[3], jnp.float32)

            def vec_body(j, c2):
                sl = pl.ds(j * _L, _L)
                uu = u_v[s, sl]
                w = w_v[s, sl]
                us = uu + (1.0 - uu) * w
                p01 = (1.0 - f0 * rw_v[s, 0, sl]) * (1.0 - f1 * rw_v[s, 1, sl])
                p23 = (1.0 - f2 * rw_v[s, 2, sl]) * (1.0 - f3 * rw_v[s, 3, sl])
                o_v[s, sl] = us * (p01 * p23)
                return c2

            lax.fori_loop(0, _M // _L, vec_body, 0, unroll=2)

        pending_in = [None] * _NBUF
        pending_out = [None] * _NBUF
        for c in range(_NBUF):
            pending_in[c] = issue_in(c, c)
        for c in range(_ROWS):
            s = c % _NBUF
            for cp in pending_in[s]:
                cp.wait()
            if pending_out[s] is not None:
                pending_out[s].wait()
            compute(c, s)
            cp = pltpu.make_async_copy(o_v.at[s], out_hbm.at[base + c], sout[s])
            cp.start()
            pending_out[s] = cp
            if c + _NBUF < _ROWS:
                pending_in[s] = issue_in(c + _NBUF, s)
        for s in range(_NBUF):
            pending_out[s].wait()

    return body(ww, rw, u, fg)


def _tc_body(ww_ref, rw_ref, u_ref, fg_ref, o_ref):
    u = u_ref[...]
    w = ww_ref[:, 0, :]
    us = u + (1.0 - u) * w
    fg = fg_ref[...]
    p = 1.0 - fg[:, 0][:, None] * rw_ref[:, 0, :]
    p = p * (1.0 - fg[:, 1][:, None] * rw_ref[:, 1, :])
    p = p * (1.0 - fg[:, 2][:, None] * rw_ref[:, 2, :])
    p = p * (1.0 - fg[:, 3][:, None] * rw_ref[:, 3, :])
    o_ref[...] = us * p


def _freeness_tc(ww, rw, u, fg):
    off = _KSC // _BBLK
    grid = ((_B - _KSC) // _BBLK,)
    return pl.pallas_call(
        _tc_body,
        grid=grid,
        in_specs=[
            pl.BlockSpec((_BBLK, 1, _M), lambda i: (i + off, 0, 0)),
            pl.BlockSpec((_BBLK, _R, _M), lambda i: (i + off, 0, 0)),
            pl.BlockSpec((_BBLK, _M), lambda i: (i + off, 0)),
            pl.BlockSpec((_BBLK, _R), lambda i: (i + off, 0)),
        ],
        out_specs=pl.BlockSpec((_BBLK, _M), lambda i: (i + off, 0)),
        out_shape=jax.ShapeDtypeStruct((_B, _M), jnp.float32),
    )(ww, rw, u, fg)


def kernel(inputs, prev_write_weight, prev_read_weight, prev_usage, free_gate):
    del inputs
    sc_part = _freeness_sc(prev_write_weight, prev_read_weight, prev_usage,
                           free_gate.reshape(_B * _R))
    tc_full = _freeness_tc(prev_write_weight, prev_read_weight, prev_usage,
                           free_gate)
    return lax.dynamic_update_slice(tc_full, sc_part, (0, 0))
